# baseline (speedup 1.0000x reference)
import functools

import jax
import jax.numpy as jnp
from jax import lax
from jax.experimental import pallas as pl
from jax.experimental.pallas import tpu as pltpu
from jax.experimental.pallas import tpu_sc as plsc

B = 16384
D = 128
CHUNK = 64
NLANE = 16


@functools.cache
def _build_sc_kernel():
    info = plsc.get_sparse_core_info()
    nc, ns = info.num_cores, info.num_subcores
    nw = nc * ns
    b_per_w = B // nw
    n_dma = b_per_w // CHUNK
    mesh = plsc.VectorSubcoreMesh(core_axis_name="c", subcore_axis_name="s")

    @functools.partial(
        pl.kernel,
        out_type=jax.ShapeDtypeStruct((B * D,), jnp.float32),
        mesh=mesh,
        scratch_types=[
            pltpu.VMEM((1,), jnp.int32),
            pltpu.VMEM((1, D), jnp.float32),
            pltpu.VMEM((CHUNK * D,), jnp.float32),
            pltpu.SemaphoreType.DMA,
        ],
    )
    def sc_broadcast_lookup(idx_hbm, table_hbm, out_hbm, idx_v, row_v, buf_v, sem):
        wid = lax.axis_index("s") * nc + lax.axis_index("c")
        base = wid * (b_per_w * D)
        pltpu.sync_copy(idx_hbm, idx_v)
        pltpu.async_copy(table_hbm.at[idx_v], row_v, sem).wait()
        row = [row_v[0, pl.ds(j * NLANE, NLANE)] for j in range(D // NLANE)]

        @plsc.parallel_loop(0, CHUNK, unroll=4)
        def _fill(r):
            for j in range(D // NLANE):
                buf_v[pl.ds(r * D + j * NLANE, NLANE)] = row[j]
        copies = [
            pltpu.async_copy(
                buf_v, out_hbm.at[pl.ds(base + j * (CHUNK * D), CHUNK * D)], sem
            )
            for j in range(n_dma)
        ]
        for c in copies:
            c.wait()

    return sc_broadcast_lookup


def kernel(task_id, batch_size, table):
    del batch_size
    idx = jnp.full((1,), task_id, dtype=jnp.int32)
    return _build_sc_kernel()(idx, table).reshape(B, D)


# speedup vs baseline: 1.0429x; 1.0429x over previous
import functools

import jax
import jax.numpy as jnp
from jax import lax
from jax.experimental import pallas as pl
from jax.experimental.pallas import tpu as pltpu
from jax.experimental.pallas import tpu_sc as plsc

B = 16384
D = 128
CHUNK = 64
NLANE = 16


@functools.cache
def _build_sc_kernel():
    info = plsc.get_sparse_core_info()
    ns = info.num_subcores
    n_hi = 12
    n_lo = 4
    mesh = plsc.VectorSubcoreMesh(core_axis_name="c", subcore_axis_name="s")

    @functools.partial(
        pl.kernel,
        out_type=jax.ShapeDtypeStruct((B * D,), jnp.float32),
        mesh=mesh,
        scratch_types=[
            pltpu.VMEM((1,), jnp.int32),
            pltpu.VMEM((1, D), jnp.float32),
            pltpu.VMEM((CHUNK * D,), jnp.float32),
            pltpu.SemaphoreType.DMA,
        ],
    )
    def sc_broadcast_lookup(idx_hbm, table_hbm, out_hbm, idx_v, row_v, buf_v, sem):
        s_idx = lax.axis_index("s")
        c_idx = lax.axis_index("c")
        base = jnp.where(
            c_idx == 1,
            s_idx * (n_hi * CHUNK * D),
            ns * (n_hi * CHUNK * D) + s_idx * (n_lo * CHUNK * D),
        )
        pltpu.sync_copy(idx_hbm, idx_v)
        pltpu.async_copy(table_hbm.at[idx_v], row_v, sem).wait()
        row = [row_v[0, pl.ds(j * NLANE, NLANE)] for j in range(D // NLANE)]

        @plsc.parallel_loop(0, CHUNK, unroll=4)
        def _fill(r):
            for j in range(D // NLANE):
                buf_v[pl.ds(r * D + j * NLANE, NLANE)] = row[j]

        copies = [
            pltpu.async_copy(
                buf_v, out_hbm.at[pl.ds(base + j * (CHUNK * D), CHUNK * D)], sem
            )
            for j in range(n_lo)
        ]

        @pl.when(c_idx == 1)
        def _():
            extra = [
                pltpu.async_copy(
                    buf_v,
                    out_hbm.at[pl.ds(base + j * (CHUNK * D), CHUNK * D)],
                    sem,
                )
                for j in range(n_lo, n_hi)
            ]
            for e in extra:
                e.wait()

        for c in copies:
            c.wait()

    return sc_broadcast_lookup


def kernel(task_id, batch_size, table):
    del batch_size
    idx = jnp.full((1,), task_id, dtype=jnp.int32)
    return _build_sc_kernel()(idx, table).reshape(B, D)
